# Initial kernel scaffold; baseline (speedup 1.0000x reference)
import jax
import jax.numpy as jnp
from jax.experimental import pallas as pl
from jax.experimental.pallas import tpu as pltpu

_REG_MAX = 16
_TOPK = 10
_BOX_W = 7.5
_OBJ_W = 1.0
_NEG = float("-inf")


def _loss_kernel(boxes_ref, scores_ref, targets_ref, box_out_ref, obj_out_ref):
    x = boxes_ref[0]
    n = x.shape[1]

    ds = []
    for k in range(4):
        xe = jnp.exp(x[k * _REG_MAX:(k + 1) * _REG_MAX, :])
        den = jnp.sum(xe, axis=0, keepdims=True)
        w = jax.lax.broadcasted_iota(jnp.float32, (_REG_MAX, n), 0)
        num = jnp.sum(xe * w, axis=0, keepdims=True)
        ds.append(num / den)
    l, t, r, b = ds
    b1x1 = -l
    b1y1 = -t
    b1x2 = r
    b1y2 = b

    tgt = targets_ref[0]
    b2x1 = tgt[:, 0:1]
    b2y1 = tgt[:, 1:2]
    b2x2 = tgt[:, 2:3]
    b2y2 = tgt[:, 3:4]

    eps = 1e-7
    w1 = b1x2 - b1x1
    h1 = b1y2 - b1y1
    w2 = b2x2 - b2x1
    h2 = b2y2 - b2y1
    inter_w = jnp.maximum(jnp.minimum(b1x2, b2x2) - jnp.maximum(b1x1, b2x1), 0.0)
    inter_h = jnp.maximum(jnp.minimum(b1y2, b2y2) - jnp.maximum(b1y1, b2y1), 0.0)
    inter = inter_w * inter_h
    union = w1 * h1 + w2 * h2 - inter + eps
    iou = inter / union
    cw = jnp.maximum(b1x2, b2x2) - jnp.minimum(b1x1, b2x1)
    ch = jnp.maximum(b1y2, b2y2) - jnp.minimum(b1y1, b2y1)
    c2 = cw * cw + ch * ch + eps
    sx1 = b1x1 + b1x2
    sy1 = b1y1 + b1y2
    sx2 = b2x1 + b2x2
    sy2 = b2y1 + b2y2
    rho2 = ((sx2 - sx1) ** 2 + (sy2 - sy1) ** 2) * 0.25
    a1 = jnp.arctan(w1 / (h1 + eps))
    a2 = jnp.arctan(w2 / (h2 + eps))
    v = (4.0 / (jnp.pi ** 2)) * (a2 - a1) ** 2
    alpha = v / (v - iou + (1.0 + eps))
    ciou = iou - (rho2 / c2 + v * alpha)

    lane = jax.lax.broadcasted_iota(jnp.int32, ciou.shape, 1)
    ciou = jnp.where(lane < n, ciou, _NEG)

    work = ciou
    thr = None
    for i in range(_TOPK):
        thr = jnp.max(work, axis=1, keepdims=True)
        if i < _TOPK - 1:
            work = jnp.where(work == thr, _NEG, work)

    selv = jnp.where(ciou >= thr, ciou, _NEG)
    m = jnp.max(selv, axis=0, keepdims=True)
    posf = (m > 0.0).astype(jnp.float32)
    it = jnp.maximum(m, 0.0)

    npos = jnp.sum(posf)
    sbox = jnp.sum((1.0 - it) * posf)
    box_b = jnp.where(npos > 0.0, sbox / jnp.maximum(npos, 1.0), 0.0)

    s = scores_ref[0]
    lane1 = jax.lax.broadcasted_iota(jnp.int32, s.shape, 1)
    valid = lane1 < n
    softplus = jnp.log1p(jnp.exp(-jnp.abs(s))) + jnp.maximum(s, 0.0)
    obj_b = (jnp.sum(jnp.where(valid, softplus, 0.0))
             - jnp.sum(jnp.where(valid, s * it, 0.0))) / n

    box_out_ref[0, 0] = box_b
    obj_out_ref[0, 0] = obj_b


@jax.jit
def kernel(boxes, scores, targets):
    bsz, c, n = boxes.shape
    m = targets.shape[1]
    box_b, obj_b = pl.pallas_call(
        _loss_kernel,
        grid=(bsz,),
        in_specs=[
            pl.BlockSpec((1, c, n), lambda i: (i, 0, 0)),
            pl.BlockSpec((1, 1, n), lambda i: (i, 0, 0)),
            pl.BlockSpec((1, m, 4), lambda i: (i, 0, 0)),
        ],
        out_specs=[
            pl.BlockSpec((1, 1), lambda i: (i, 0)),
            pl.BlockSpec((1, 1), lambda i: (i, 0)),
        ],
        out_shape=[
            jax.ShapeDtypeStruct((bsz, 1), jnp.float32),
            jax.ShapeDtypeStruct((bsz, 1), jnp.float32),
        ],
        compiler_params=pltpu.CompilerParams(
            dimension_semantics=("arbitrary",),
        ),
    )(boxes, scores, targets)
    tb = jnp.sum(box_b)
    to = jnp.sum(obj_b)
    total = (_BOX_W * tb + _OBJ_W * to) / bsz
    return total, jax.lax.stop_gradient(tb), jax.lax.stop_gradient(to)


# speedup vs baseline: 38.2914x; 38.2914x over previous
import jax
import jax.numpy as jnp
from jax.experimental import pallas as pl
from jax.experimental.pallas import tpu as pltpu

_REG_MAX = 16
_TOPK = 10
_BOX_W = 7.5
_OBJ_W = 1.0
_NEG = float("-inf")

_ATAN_COEF = (
    0.9999999999902919, -0.33333332995051296, 0.19999980353689645,
    -0.14285262492495704, 0.11105656189675474, -0.09051137251409,
    0.07502231366742305, -0.06038548449194854, 0.04390286868997824,
    -0.026271574631780946, 0.011602323441057973, -0.003261486111460649,
    0.00043016480682746657,
)
_HALF_PI = 1.5707963267948966


def _atan_pos(z):
    inv = z > 1.0
    x = jnp.where(inv, 1.0 / jnp.maximum(z, 1e-30), z)
    t = x * x
    acc = jnp.full_like(t, _ATAN_COEF[-1])
    for c in _ATAN_COEF[-2::-1]:
        acc = acc * t + c
    r = x * acc
    return jnp.where(inv, _HALF_PI - r, r)


def _loss_kernel(boxes_ref, scores_ref, targets_ref, box_out_ref, obj_out_ref):
    x = boxes_ref[0]
    n = x.shape[1]

    ds = []
    for k in range(4):
        xe = jnp.exp(x[k * _REG_MAX:(k + 1) * _REG_MAX, :])
        den = jnp.sum(xe, axis=0, keepdims=True)
        w = jax.lax.broadcasted_iota(jnp.int32, (_REG_MAX, n), 0).astype(jnp.float32)
        num = jnp.sum(xe * w, axis=0, keepdims=True)
        ds.append(num / den)
    l, t, r, b = ds
    b1x1 = -l
    b1y1 = -t
    b1x2 = r
    b1y2 = b

    tgt = targets_ref[0]
    b2x1 = tgt[:, 0:1]
    b2y1 = tgt[:, 1:2]
    b2x2 = tgt[:, 2:3]
    b2y2 = tgt[:, 3:4]

    eps = 1e-7
    w1 = b1x2 - b1x1
    h1 = b1y2 - b1y1
    w2 = b2x2 - b2x1
    h2 = b2y2 - b2y1
    inter_w = jnp.maximum(jnp.minimum(b1x2, b2x2) - jnp.maximum(b1x1, b2x1), 0.0)
    inter_h = jnp.maximum(jnp.minimum(b1y2, b2y2) - jnp.maximum(b1y1, b2y1), 0.0)
    inter = inter_w * inter_h
    union = w1 * h1 + w2 * h2 - inter + eps
    iou = inter / union
    cw = jnp.maximum(b1x2, b2x2) - jnp.minimum(b1x1, b2x1)
    ch = jnp.maximum(b1y2, b2y2) - jnp.minimum(b1y1, b2y1)
    c2 = cw * cw + ch * ch + eps
    sx1 = b1x1 + b1x2
    sy1 = b1y1 + b1y2
    sx2 = b2x1 + b2x2
    sy2 = b2y1 + b2y2
    rho2 = ((sx2 - sx1) ** 2 + (sy2 - sy1) ** 2) * 0.25
    a1 = _atan_pos(w1 / (h1 + eps))
    a2 = _atan_pos(w2 / (h2 + eps))
    v = (4.0 / (jnp.pi ** 2)) * (a2 - a1) ** 2
    alpha = v / (v - iou + (1.0 + eps))
    ciou = iou - (rho2 / c2 + v * alpha)

    lane = jax.lax.broadcasted_iota(jnp.int32, ciou.shape, 1)
    ciou = jnp.where(lane < n, ciou, _NEG)

    work = ciou
    thr = None
    for i in range(_TOPK):
        thr = jnp.max(work, axis=1, keepdims=True)
        if i < _TOPK - 1:
            work = jnp.where(work == thr, _NEG, work)

    selv = jnp.where(ciou >= thr, ciou, _NEG)
    m = jnp.max(selv, axis=0, keepdims=True)
    posf = (m > 0.0).astype(jnp.float32)
    it = jnp.maximum(m, 0.0)

    npos = jnp.sum(posf)
    sbox = jnp.sum((1.0 - it) * posf)
    box_b = jnp.where(npos > 0.0, sbox / jnp.maximum(npos, 1.0), 0.0)

    s = scores_ref[0]
    lane1 = jax.lax.broadcasted_iota(jnp.int32, s.shape, 1)
    valid = lane1 < n
    softplus = jnp.log1p(jnp.exp(-jnp.abs(s))) + jnp.maximum(s, 0.0)
    obj_b = (jnp.sum(jnp.where(valid, softplus, 0.0))
             - jnp.sum(jnp.where(valid, s * it, 0.0))) / n

    box_out_ref[0] = jnp.reshape(box_b, (1, 1))
    obj_out_ref[0] = jnp.reshape(obj_b, (1, 1))


@jax.jit
def kernel(boxes, scores, targets):
    bsz, c, n = boxes.shape
    m = targets.shape[1]
    box_b, obj_b = pl.pallas_call(
        _loss_kernel,
        grid=(bsz,),
        in_specs=[
            pl.BlockSpec((1, c, n), lambda i: (i, 0, 0)),
            pl.BlockSpec((1, 1, n), lambda i: (i, 0, 0)),
            pl.BlockSpec((1, m, 4), lambda i: (i, 0, 0)),
        ],
        out_specs=[
            pl.BlockSpec((1, 1, 1), lambda i: (i, 0, 0)),
            pl.BlockSpec((1, 1, 1), lambda i: (i, 0, 0)),
        ],
        out_shape=[
            jax.ShapeDtypeStruct((bsz, 1, 1), jnp.float32),
            jax.ShapeDtypeStruct((bsz, 1, 1), jnp.float32),
        ],
        compiler_params=pltpu.CompilerParams(
            dimension_semantics=("arbitrary",),
        ),
    )(boxes, scores, targets)
    tb = jnp.sum(box_b)
    to = jnp.sum(obj_b)
    total = (_BOX_W * tb + _OBJ_W * to) / bsz
    return total, jax.lax.stop_gradient(tb), jax.lax.stop_gradient(to)
